# baseline (speedup 1.0000x reference)
import functools

import jax
import jax.numpy as jnp
from jax import lax
from jax.experimental import pallas as pl
from jax.experimental.pallas import tpu as pltpu
from jax.experimental.pallas import tpu_sc as plsc

B = 8
N = 4096
NPOINT = 1024
NSAMPLE = 32
CIN = 128
EPS = 1e-5
BIGF = 1e10
CT = 512
TM = 2048
PN = B * NPOINT * NSAMPLE
NW = 32
PERW = PN // NW
CH = 128


def _fps_body(xyz_ref, idx_ref, nxyz_ref):
    xs = xyz_ref[0]
    ys = xyz_ref[1]
    zs = xyz_ref[2]
    iota = jax.lax.broadcasted_iota(jnp.int32, (B, N), 1)
    row_iota = jax.lax.broadcasted_iota(jnp.int32, (B, NPOINT), 0)
    iota_np = jax.lax.broadcasted_iota(jnp.int32, (B, NPOINT), 1)

    def body(i, carry):
        dist, far, oidx, ox, oy, oz = carry
        oh = iota == far
        cx = jnp.sum(jnp.where(oh, xs, 0.0), axis=1, keepdims=True)
        cy = jnp.sum(jnp.where(oh, ys, 0.0), axis=1, keepdims=True)
        cz = jnp.sum(jnp.where(oh, zs, 0.0), axis=1, keepdims=True)
        sel = (iota_np == i) & (row_iota >= 0)
        oidx = oidx + jnp.where(sel, jnp.broadcast_to(far, (B, NPOINT)), 0)
        ox = ox + jnp.where(sel, jnp.broadcast_to(cx, (B, NPOINT)), 0.0)
        oy = oy + jnp.where(sel, jnp.broadcast_to(cy, (B, NPOINT)), 0.0)
        oz = oz + jnp.where(sel, jnp.broadcast_to(cz, (B, NPOINT)), 0.0)
        d = (xs - cx) ** 2 + (ys - cy) ** 2 + (zs - cz) ** 2
        dist = jnp.minimum(dist, d)
        m = jnp.max(dist, axis=1, keepdims=True)
        far2 = jnp.min(jnp.where(dist == m, iota, N), axis=1,
                       keepdims=True).astype(jnp.int32)
        return dist, far2, oidx, ox, oy, oz

    dist0 = jnp.full((B, N), BIGF, jnp.float32)
    far0 = jnp.zeros((B, 1), jnp.int32)
    zf = jnp.zeros((B, NPOINT), jnp.float32)
    zi = jnp.zeros((B, NPOINT), jnp.int32)
    _, _, oidx, ox, oy, oz = jax.lax.fori_loop(
        0, NPOINT, body, (dist0, far0, zi, zf, zf, zf))
    idx_ref[...] = oidx
    nxyz_ref[:, 0, :] = ox
    nxyz_ref[:, 1, :] = oy
    nxyz_ref[:, 2, :] = oz


def _fps(xyz_t):
    return pl.pallas_call(
        _fps_body,
        out_shape=[
            jax.ShapeDtypeStruct((B, NPOINT), jnp.int32),
            jax.ShapeDtypeStruct((B, 3, NPOINT), jnp.float32),
        ],
    )(xyz_t)


def _knn_body(xyz_ref, nxyz_ref, idx_ref, d_scr, i_scr):
    xmat = xyz_ref[0]
    cmat = nxyz_ref[0]
    mm = jnp.dot(xmat, cmat, preferred_element_type=jnp.float32)
    d = -2.0 * mm
    d = d + jnp.sum(xmat * xmat, axis=1, keepdims=True)
    d = d + jnp.sum(cmat * cmat, axis=0, keepdims=True)
    d_scr[...] = d
    iota = jax.lax.broadcasted_iota(jnp.int32, (N, CT), 0)

    def ext(k, carry):
        mprev, aprev = carry
        dv = d_scr[...]
        valid = (dv > mprev) | ((dv == mprev) & (iota > aprev))
        dm = jnp.where(valid, dv, BIGF)
        m = jnp.min(dm, axis=0, keepdims=True)
        am = jnp.min(jnp.where(dm == m, iota, N), axis=0,
                     keepdims=True).astype(jnp.int32)
        i_scr[pl.ds(k, 1), :] = am
        return m, am

    jax.lax.fori_loop(
        0, NSAMPLE, ext,
        (jnp.full((1, CT), -BIGF, jnp.float32),
         jnp.full((1, CT), -1, jnp.int32)))
    off = pl.program_id(0) * N
    idx_ref[0] = jnp.transpose(i_scr[...], (1, 0)) + off


def _knn(xyz, nxyz_b):
    return pl.pallas_call(
        _knn_body,
        grid=(B, NPOINT // CT),
        in_specs=[
            pl.BlockSpec((1, N, 3), lambda b, t: (b, 0, 0)),
            pl.BlockSpec((1, 3, CT), lambda b, t: (b, 0, t)),
        ],
        out_specs=pl.BlockSpec((1, CT, NSAMPLE), lambda b, t: (b, t, 0)),
        out_shape=jax.ShapeDtypeStruct((B, NPOINT, NSAMPLE), jnp.int32),
        scratch_shapes=[pltpu.VMEM((N, CT), jnp.float32),
                        pltpu.VMEM((NSAMPLE, CT), jnp.int32)],
    )(xyz, nxyz_b)


def _h_body(f_ref, x_ref, wf_ref, wx_ref, h_ref):
    f = f_ref[0]
    h = jax.lax.dot_general(
        f, wf_ref[...], (((0,), (0,)), ((), ())),
        preferred_element_type=jnp.float32)
    h = h + jnp.dot(x_ref[0], wx_ref[...],
                    preferred_element_type=jnp.float32)
    h_ref[0] = h


def _h_transform(features, xyz, w0ft, w0xt):
    return pl.pallas_call(
        _h_body,
        grid=(B, N // 512),
        in_specs=[
            pl.BlockSpec((1, CIN, 512), lambda b, t: (b, 0, t)),
            pl.BlockSpec((1, 512, 3), lambda b, t: (b, t, 0)),
            pl.BlockSpec((CIN, CIN), lambda b, t: (0, 0)),
            pl.BlockSpec((3, CIN), lambda b, t: (0, 0)),
        ],
        out_specs=pl.BlockSpec((1, 512, CIN), lambda b, t: (b, t, 0)),
        out_shape=jax.ShapeDtypeStruct((B, N, CIN), jnp.float32),
    )(features, xyz, w0ft, w0xt)


def _gather_sc(idx_flat, htab):
    mesh = plsc.VectorSubcoreMesh(core_axis_name="c", subcore_axis_name="s")

    nbuf = 4

    @functools.partial(
        pl.kernel, mesh=mesh,
        out_type=jax.ShapeDtypeStruct((PN, CIN), jnp.float32),
        scratch_types=(
            [pltpu.VMEM((PERW,), jnp.int32)]
            + [pltpu.VMEM((CH, CIN), jnp.float32)] * nbuf
            + [pltpu.SemaphoreType.DMA] * (2 * nbuf)
        ),
    )
    def k(idx_hbm, h_hbm, out_hbm, idxb, *bufs):
        grows = bufs[:nbuf]
        gsems = bufs[nbuf:2 * nbuf]
        wsems = bufs[2 * nbuf:]
        wid = lax.axis_index("s") * 2 + lax.axis_index("c")
        base = wid * PERW
        pltpu.sync_copy(idx_hbm.at[pl.ds(base, PERW)], idxb)

        def gth(q, _):
            offs = [pl.multiple_of((q * nbuf + i) * CH, CH)
                    for i in range(nbuf)]
            cps = [pltpu.async_copy(h_hbm.at[idxb.at[pl.ds(offs[i], CH)]],
                                    grows[i], gsems[i])
                   for i in range(nbuf)]
            wps = []
            for i in range(nbuf):
                cps[i].wait()
                wps.append(pltpu.async_copy(
                    grows[i], out_hbm.at[pl.ds(base + offs[i], CH)],
                    wsems[i]))
            for w in wps:
                w.wait()
            return 0

        jax.lax.fori_loop(0, PERW // CH // nbuf, gth, 0)

    return k(idx_flat, htab)


def _acc_stats(y, s_ref, q_ref):
    ps = jnp.sum(y, axis=0, keepdims=True)
    pq = jnp.sum(y * y, axis=0, keepdims=True)

    @pl.when(pl.program_id(0) == 0)
    def _():
        s_ref[...] = ps
        q_ref[...] = pq

    @pl.when(pl.program_id(0) != 0)
    def _():
        s_ref[...] = s_ref[...] + ps
        q_ref[...] = q_ref[...] + pq


def _mlp0_body(hg_ref, c_ref, wx_ref, b_ref, y_ref, s_ref, q_ref):
    cp = jnp.dot(c_ref[...], wx_ref[...],
                 preferred_element_type=jnp.float32)
    crep = jnp.broadcast_to(cp[:, None, :],
                            (TM // NSAMPLE, NSAMPLE, cp.shape[-1]))
    crep = crep.reshape(TM, cp.shape[-1])
    y = hg_ref[...] - crep
    y = y + b_ref[...]
    y_ref[...] = y
    _acc_stats(y, s_ref, q_ref)


def _norm_relu(y, s_ref, q_ref, g_ref, be_ref):
    mean = s_ref[...] / PN
    var = q_ref[...] / PN - mean * mean
    xn = (y - mean) / jnp.sqrt(var + EPS) * g_ref[...] + be_ref[...]
    return jnp.maximum(xn, 0.0)


def _mlp_mid_body(y0_ref, s0_ref, q0_ref, g_ref, be_ref, w_ref, b_ref,
                  y_ref, s_ref, q_ref):
    x = _norm_relu(y0_ref[...], s0_ref, q0_ref, g_ref, be_ref)
    y = jnp.dot(x, w_ref[...], preferred_element_type=jnp.float32) + b_ref[...]
    y_ref[...] = y
    _acc_stats(y, s_ref, q_ref)


def _mlp_out_body(y2_ref, s2_ref, q2_ref, g_ref, be_ref, o_ref):
    x = _norm_relu(y2_ref[...], s2_ref, q2_ref, g_ref, be_ref)
    xr = x.reshape(TM // NSAMPLE, NSAMPLE, x.shape[-1])
    o_ref[...] = jnp.max(xr, axis=1)


def _row_spec(c):
    return pl.BlockSpec((TM, c), lambda s: (s, 0))


def _full_spec(shape):
    return pl.BlockSpec(shape, lambda s: tuple(0 for _ in shape))


def _mlp(hg, cpad, params):
    (w0, b0, g0, be0), (w1, b1, g1, be1), (w2, b2, g2, be2) = params
    steps = PN // TM
    c1, c2 = 128, 256
    w0xp = jnp.zeros((8, c1), jnp.float32).at[:3, :].set(w0[:, :3].T)
    w1t = jnp.transpose(w1)
    w2t = jnp.transpose(w2)
    r = lambda v: v.reshape(1, -1)

    y0, s0, q0 = pl.pallas_call(
        _mlp0_body,
        grid=(steps,),
        in_specs=[
            _row_spec(CIN),
            pl.BlockSpec((TM // NSAMPLE, 8), lambda s: (s, 0)),
            _full_spec((8, c1)), _full_spec((1, c1)),
        ],
        out_specs=[
            _row_spec(c1),
            pl.BlockSpec((1, c1), lambda s: (0, 0)),
            pl.BlockSpec((1, c1), lambda s: (0, 0)),
        ],
        out_shape=[
            jax.ShapeDtypeStruct((PN, c1), jnp.float32),
            jax.ShapeDtypeStruct((1, c1), jnp.float32),
            jax.ShapeDtypeStruct((1, c1), jnp.float32),
        ],
    )(hg, cpad, w0xp, r(b0))

    def mid(y, s, q, g, be, wt, b, cout):
        return pl.pallas_call(
            _mlp_mid_body,
            grid=(steps,),
            in_specs=[
                _row_spec(y.shape[-1]),
                _full_spec((1, y.shape[-1])), _full_spec((1, y.shape[-1])),
                _full_spec((1, y.shape[-1])), _full_spec((1, y.shape[-1])),
                _full_spec((y.shape[-1], cout)), _full_spec((1, cout)),
            ],
            out_specs=[
                _row_spec(cout),
                pl.BlockSpec((1, cout), lambda s: (0, 0)),
                pl.BlockSpec((1, cout), lambda s: (0, 0)),
            ],
            out_shape=[
                jax.ShapeDtypeStruct((PN, cout), jnp.float32),
                jax.ShapeDtypeStruct((1, cout), jnp.float32),
                jax.ShapeDtypeStruct((1, cout), jnp.float32),
            ],
        )(y, s, q, r(g), r(be), wt, b)

    y1, s1, q1 = mid(y0, s0, q0, g0, be0, w1t, r(b1), c1)
    y2, s2, q2 = mid(y1, s1, q1, g1, be1, w2t, r(b2), c2)

    out = pl.pallas_call(
        _mlp_out_body,
        grid=(steps,),
        in_specs=[
            _row_spec(c2),
            _full_spec((1, c2)), _full_spec((1, c2)),
            _full_spec((1, c2)), _full_spec((1, c2)),
        ],
        out_specs=pl.BlockSpec((TM // NSAMPLE, c2), lambda s: (s, 0)),
        out_shape=jax.ShapeDtypeStruct((B * NPOINT, c2), jnp.float32),
    )(y2, s2, q2, r(g2), r(be2))
    return out


def kernel(xyz, features, W0, b0, g0, be0, W1, b1, g1, be1, W2, b2, g2, be2):
    xyz_t = jnp.transpose(xyz, (2, 0, 1))
    _, nxyz_b = _fps(xyz_t)
    new_xyz = jnp.transpose(nxyz_b, (0, 2, 1))

    htab = _h_transform(features, xyz, jnp.transpose(W0[:, 3:]),
                        jnp.transpose(W0[:, :3])).reshape(B * N, CIN)

    idx = _knn(xyz, nxyz_b)
    hg = _gather_sc(idx.reshape(PN), htab)

    cpad = jnp.pad(new_xyz, ((0, 0), (0, 0), (0, 5))).reshape(B * NPOINT, 8)
    params = [(W0, b0, g0, be0), (W1, b1, g1, be1), (W2, b2, g2, be2)]
    outf = _mlp(hg, cpad, params)
    new_features = jnp.transpose(outf.reshape(B, NPOINT, 256), (0, 2, 1))
    return new_xyz, new_features


# speedup vs baseline: 1.0238x; 1.0238x over previous
import functools

import jax
import jax.numpy as jnp
from jax import lax
from jax.experimental import pallas as pl
from jax.experimental.pallas import tpu as pltpu
from jax.experimental.pallas import tpu_sc as plsc

B = 8
N = 4096
NPOINT = 1024
NSAMPLE = 32
CIN = 128
EPS = 1e-5
BIGF = 1e10
CT = 256
TM = 2048
PN = B * NPOINT * NSAMPLE
NW = 32
PERW = PN // NW
CH = 128


def _fps_body(xyz_ref, idx_ref, nxyz_ref):
    xs = xyz_ref[0]
    ys = xyz_ref[1]
    zs = xyz_ref[2]
    iota = jax.lax.broadcasted_iota(jnp.int32, (B, N), 1)
    row_iota = jax.lax.broadcasted_iota(jnp.int32, (B, NPOINT), 0)
    iota_np = jax.lax.broadcasted_iota(jnp.int32, (B, NPOINT), 1)

    def body(i, carry):
        dist, far, oidx, ox, oy, oz = carry
        oh = iota == far
        cx = jnp.sum(jnp.where(oh, xs, 0.0), axis=1, keepdims=True)
        cy = jnp.sum(jnp.where(oh, ys, 0.0), axis=1, keepdims=True)
        cz = jnp.sum(jnp.where(oh, zs, 0.0), axis=1, keepdims=True)
        sel = (iota_np == i) & (row_iota >= 0)
        oidx = oidx + jnp.where(sel, jnp.broadcast_to(far, (B, NPOINT)), 0)
        ox = ox + jnp.where(sel, jnp.broadcast_to(cx, (B, NPOINT)), 0.0)
        oy = oy + jnp.where(sel, jnp.broadcast_to(cy, (B, NPOINT)), 0.0)
        oz = oz + jnp.where(sel, jnp.broadcast_to(cz, (B, NPOINT)), 0.0)
        d = (xs - cx) ** 2 + (ys - cy) ** 2 + (zs - cz) ** 2
        dist = jnp.minimum(dist, d)
        m = jnp.max(dist, axis=1, keepdims=True)
        far2 = jnp.min(jnp.where(dist == m, iota, N), axis=1,
                       keepdims=True).astype(jnp.int32)
        return dist, far2, oidx, ox, oy, oz

    dist0 = jnp.full((B, N), BIGF, jnp.float32)
    far0 = jnp.zeros((B, 1), jnp.int32)
    zf = jnp.zeros((B, NPOINT), jnp.float32)
    zi = jnp.zeros((B, NPOINT), jnp.int32)
    _, _, oidx, ox, oy, oz = jax.lax.fori_loop(
        0, NPOINT, body, (dist0, far0, zi, zf, zf, zf))
    idx_ref[...] = oidx
    nxyz_ref[:, 0, :] = ox
    nxyz_ref[:, 1, :] = oy
    nxyz_ref[:, 2, :] = oz


def _fps(xyz_t):
    return pl.pallas_call(
        _fps_body,
        out_shape=[
            jax.ShapeDtypeStruct((B, NPOINT), jnp.int32),
            jax.ShapeDtypeStruct((B, 3, NPOINT), jnp.float32),
        ],
    )(xyz_t)


def _knn_body(xyz_ref, nxyz_ref, idx_ref, d_scr, i_scr):
    xmat = xyz_ref[0]
    cmat = nxyz_ref[0]
    mm = jnp.dot(xmat, cmat, preferred_element_type=jnp.float32)
    d = -2.0 * mm
    d = d + jnp.sum(xmat * xmat, axis=1, keepdims=True)
    d = d + jnp.sum(cmat * cmat, axis=0, keepdims=True)
    d_scr[...] = d
    iota = jax.lax.broadcasted_iota(jnp.int32, (N, CT), 0)

    def ext(k, carry):
        mprev, aprev = carry
        dv = d_scr[...]
        valid = (dv > mprev) | ((dv == mprev) & (iota > aprev))
        dm = jnp.where(valid, dv, BIGF)
        m = jnp.min(dm, axis=0, keepdims=True)
        am = jnp.min(jnp.where(dm == m, iota, N), axis=0,
                     keepdims=True).astype(jnp.int32)
        i_scr[pl.ds(k, 1), :] = am
        return m, am

    jax.lax.fori_loop(
        0, NSAMPLE, ext,
        (jnp.full((1, CT), -BIGF, jnp.float32),
         jnp.full((1, CT), -1, jnp.int32)))
    off = pl.program_id(0) * N
    idx_ref[0] = jnp.transpose(i_scr[...], (1, 0)) + off


def _knn(xyz, nxyz_b):
    return pl.pallas_call(
        _knn_body,
        grid=(B, NPOINT // CT),
        in_specs=[
            pl.BlockSpec((1, N, 3), lambda b, t: (b, 0, 0)),
            pl.BlockSpec((1, 3, CT), lambda b, t: (b, 0, t)),
        ],
        out_specs=pl.BlockSpec((1, CT, NSAMPLE), lambda b, t: (b, t, 0)),
        out_shape=jax.ShapeDtypeStruct((B, NPOINT, NSAMPLE), jnp.int32),
        scratch_shapes=[pltpu.VMEM((N, CT), jnp.float32),
                        pltpu.VMEM((NSAMPLE, CT), jnp.int32)],
    )(xyz, nxyz_b)


def _h_body(f_ref, x_ref, wf_ref, wx_ref, h_ref):
    f = f_ref[0]
    h = jax.lax.dot_general(
        f, wf_ref[...], (((0,), (0,)), ((), ())),
        preferred_element_type=jnp.float32)
    h = h + jnp.dot(x_ref[0], wx_ref[...],
                    preferred_element_type=jnp.float32)
    h_ref[0] = h


def _h_transform(features, xyz, w0ft, w0xt):
    return pl.pallas_call(
        _h_body,
        grid=(B, N // 512),
        in_specs=[
            pl.BlockSpec((1, CIN, 512), lambda b, t: (b, 0, t)),
            pl.BlockSpec((1, 512, 3), lambda b, t: (b, t, 0)),
            pl.BlockSpec((CIN, CIN), lambda b, t: (0, 0)),
            pl.BlockSpec((3, CIN), lambda b, t: (0, 0)),
        ],
        out_specs=pl.BlockSpec((1, 512, CIN), lambda b, t: (b, t, 0)),
        out_shape=jax.ShapeDtypeStruct((B, N, CIN), jnp.float32),
    )(features, xyz, w0ft, w0xt)


def _gather_sc(idx_flat, htab):
    mesh = plsc.VectorSubcoreMesh(core_axis_name="c", subcore_axis_name="s")

    nbuf = 4

    @functools.partial(
        pl.kernel, mesh=mesh,
        out_type=jax.ShapeDtypeStruct((PN, CIN), jnp.float32),
        scratch_types=(
            [pltpu.VMEM((PERW,), jnp.int32)]
            + [pltpu.VMEM((CH, CIN), jnp.float32)] * nbuf
            + [pltpu.SemaphoreType.DMA] * (2 * nbuf)
        ),
    )
    def k(idx_hbm, h_hbm, out_hbm, idxb, *bufs):
        grows = bufs[:nbuf]
        gsems = bufs[nbuf:2 * nbuf]
        wsems = bufs[2 * nbuf:]
        wid = lax.axis_index("s") * 2 + lax.axis_index("c")
        base = wid * PERW
        pltpu.sync_copy(idx_hbm.at[pl.ds(base, PERW)], idxb)

        def gth(q, _):
            offs = [pl.multiple_of((q * nbuf + i) * CH, CH)
                    for i in range(nbuf)]
            cps = [pltpu.async_copy(h_hbm.at[idxb.at[pl.ds(offs[i], CH)]],
                                    grows[i], gsems[i])
                   for i in range(nbuf)]
            wps = []
            for i in range(nbuf):
                cps[i].wait()
                wps.append(pltpu.async_copy(
                    grows[i], out_hbm.at[pl.ds(base + offs[i], CH)],
                    wsems[i]))
            for w in wps:
                w.wait()
            return 0

        jax.lax.fori_loop(0, PERW // CH // nbuf, gth, 0)

    return k(idx_flat, htab)


def _acc_stats(y, s_ref, q_ref):
    ps = jnp.sum(y, axis=0, keepdims=True)
    pq = jnp.sum(y * y, axis=0, keepdims=True)

    @pl.when(pl.program_id(0) == 0)
    def _():
        s_ref[...] = ps
        q_ref[...] = pq

    @pl.when(pl.program_id(0) != 0)
    def _():
        s_ref[...] = s_ref[...] + ps
        q_ref[...] = q_ref[...] + pq


def _mlp0_body(hg_ref, c_ref, wx_ref, b_ref, y_ref, s_ref, q_ref):
    cp = jnp.dot(c_ref[...], wx_ref[...],
                 preferred_element_type=jnp.float32)
    crep = jnp.broadcast_to(cp[:, None, :],
                            (TM // NSAMPLE, NSAMPLE, cp.shape[-1]))
    crep = crep.reshape(TM, cp.shape[-1])
    y = hg_ref[...] - crep
    y = y + b_ref[...]
    y_ref[...] = y
    _acc_stats(y, s_ref, q_ref)


def _norm_relu(y, s_ref, q_ref, g_ref, be_ref):
    mean = s_ref[...] / PN
    var = q_ref[...] / PN - mean * mean
    xn = (y - mean) / jnp.sqrt(var + EPS) * g_ref[...] + be_ref[...]
    return jnp.maximum(xn, 0.0)


def _mlp_mid_body(y0_ref, s0_ref, q0_ref, g_ref, be_ref, w_ref, b_ref,
                  y_ref, s_ref, q_ref):
    x = _norm_relu(y0_ref[...], s0_ref, q0_ref, g_ref, be_ref)
    y = jnp.dot(x, w_ref[...], preferred_element_type=jnp.float32) + b_ref[...]
    y_ref[...] = y
    _acc_stats(y, s_ref, q_ref)


def _mlp_out_body(y2_ref, s2_ref, q2_ref, g_ref, be_ref, o_ref):
    x = _norm_relu(y2_ref[...], s2_ref, q2_ref, g_ref, be_ref)
    xr = x.reshape(TM // NSAMPLE, NSAMPLE, x.shape[-1])
    o_ref[...] = jnp.max(xr, axis=1)


def _row_spec(c):
    return pl.BlockSpec((TM, c), lambda s: (s, 0))


def _full_spec(shape):
    return pl.BlockSpec(shape, lambda s: tuple(0 for _ in shape))


def _mlp(hg, cpad, params):
    (w0, b0, g0, be0), (w1, b1, g1, be1), (w2, b2, g2, be2) = params
    steps = PN // TM
    c1, c2 = 128, 256
    w0xp = jnp.zeros((8, c1), jnp.float32).at[:3, :].set(w0[:, :3].T)
    w1t = jnp.transpose(w1)
    w2t = jnp.transpose(w2)
    r = lambda v: v.reshape(1, -1)

    y0, s0, q0 = pl.pallas_call(
        _mlp0_body,
        grid=(steps,),
        in_specs=[
            _row_spec(CIN),
            pl.BlockSpec((TM // NSAMPLE, 8), lambda s: (s, 0)),
            _full_spec((8, c1)), _full_spec((1, c1)),
        ],
        out_specs=[
            _row_spec(c1),
            pl.BlockSpec((1, c1), lambda s: (0, 0)),
            pl.BlockSpec((1, c1), lambda s: (0, 0)),
        ],
        out_shape=[
            jax.ShapeDtypeStruct((PN, c1), jnp.float32),
            jax.ShapeDtypeStruct((1, c1), jnp.float32),
            jax.ShapeDtypeStruct((1, c1), jnp.float32),
        ],
    )(hg, cpad, w0xp, r(b0))

    def mid(y, s, q, g, be, wt, b, cout):
        return pl.pallas_call(
            _mlp_mid_body,
            grid=(steps,),
            in_specs=[
                _row_spec(y.shape[-1]),
                _full_spec((1, y.shape[-1])), _full_spec((1, y.shape[-1])),
                _full_spec((1, y.shape[-1])), _full_spec((1, y.shape[-1])),
                _full_spec((y.shape[-1], cout)), _full_spec((1, cout)),
            ],
            out_specs=[
                _row_spec(cout),
                pl.BlockSpec((1, cout), lambda s: (0, 0)),
                pl.BlockSpec((1, cout), lambda s: (0, 0)),
            ],
            out_shape=[
                jax.ShapeDtypeStruct((PN, cout), jnp.float32),
                jax.ShapeDtypeStruct((1, cout), jnp.float32),
                jax.ShapeDtypeStruct((1, cout), jnp.float32),
            ],
        )(y, s, q, r(g), r(be), wt, b)

    y1, s1, q1 = mid(y0, s0, q0, g0, be0, w1t, r(b1), c1)
    y2, s2, q2 = mid(y1, s1, q1, g1, be1, w2t, r(b2), c2)

    out = pl.pallas_call(
        _mlp_out_body,
        grid=(steps,),
        in_specs=[
            _row_spec(c2),
            _full_spec((1, c2)), _full_spec((1, c2)),
            _full_spec((1, c2)), _full_spec((1, c2)),
        ],
        out_specs=pl.BlockSpec((TM // NSAMPLE, c2), lambda s: (s, 0)),
        out_shape=jax.ShapeDtypeStruct((B * NPOINT, c2), jnp.float32),
    )(y2, s2, q2, r(g2), r(be2))
    return out


def kernel(xyz, features, W0, b0, g0, be0, W1, b1, g1, be1, W2, b2, g2, be2):
    xyz_t = jnp.transpose(xyz, (2, 0, 1))
    _, nxyz_b = _fps(xyz_t)
    new_xyz = jnp.transpose(nxyz_b, (0, 2, 1))

    htab = _h_transform(features, xyz, jnp.transpose(W0[:, 3:]),
                        jnp.transpose(W0[:, :3])).reshape(B * N, CIN)

    idx = _knn(xyz, nxyz_b)
    hg = _gather_sc(idx.reshape(PN), htab)

    cpad = jnp.pad(new_xyz, ((0, 0), (0, 0), (0, 5))).reshape(B * NPOINT, 8)
    params = [(W0, b0, g0, be0), (W1, b1, g1, be1), (W2, b2, g2, be2)]
    outf = _mlp(hg, cpad, params)
    new_features = jnp.transpose(outf.reshape(B, NPOINT, 256), (0, 2, 1))
    return new_xyz, new_features


# speedup vs baseline: 1.0815x; 1.0564x over previous
import functools

import jax
import jax.numpy as jnp
from jax import lax
from jax.experimental import pallas as pl
from jax.experimental.pallas import tpu as pltpu
from jax.experimental.pallas import tpu_sc as plsc

B = 8
N = 4096
NPOINT = 1024
NSAMPLE = 32
CIN = 128
EPS = 1e-5
BIGF = 1e10
CT = 256
TM = 2048
PN = B * NPOINT * NSAMPLE
NW = 32
PERW = PN // NW
CH = 128


def _fps_body(xyz_ref, idx_ref, nxyz_ref):
    xs = xyz_ref[0]
    ys = xyz_ref[1]
    zs = xyz_ref[2]
    iota = jax.lax.broadcasted_iota(jnp.int32, (B, N), 1)
    row_iota = jax.lax.broadcasted_iota(jnp.int32, (B, NPOINT), 0)
    iota_np = jax.lax.broadcasted_iota(jnp.int32, (B, NPOINT), 1)

    def body(i, carry):
        dist, far, oidx, ox, oy, oz = carry
        oh = iota == far
        cx = jnp.sum(jnp.where(oh, xs, 0.0), axis=1, keepdims=True)
        cy = jnp.sum(jnp.where(oh, ys, 0.0), axis=1, keepdims=True)
        cz = jnp.sum(jnp.where(oh, zs, 0.0), axis=1, keepdims=True)
        sel = (iota_np == i) & (row_iota >= 0)
        oidx = oidx + jnp.where(sel, jnp.broadcast_to(far, (B, NPOINT)), 0)
        ox = ox + jnp.where(sel, jnp.broadcast_to(cx, (B, NPOINT)), 0.0)
        oy = oy + jnp.where(sel, jnp.broadcast_to(cy, (B, NPOINT)), 0.0)
        oz = oz + jnp.where(sel, jnp.broadcast_to(cz, (B, NPOINT)), 0.0)
        d = (xs - cx) ** 2 + (ys - cy) ** 2 + (zs - cz) ** 2
        dist = jnp.minimum(dist, d)
        m = jnp.max(dist, axis=1, keepdims=True)
        far2 = jnp.min(jnp.where(dist == m, iota, N), axis=1,
                       keepdims=True).astype(jnp.int32)
        return dist, far2, oidx, ox, oy, oz

    dist0 = jnp.full((B, N), BIGF, jnp.float32)
    far0 = jnp.zeros((B, 1), jnp.int32)
    zf = jnp.zeros((B, NPOINT), jnp.float32)
    zi = jnp.zeros((B, NPOINT), jnp.int32)
    _, _, oidx, ox, oy, oz = jax.lax.fori_loop(
        0, NPOINT, body, (dist0, far0, zi, zf, zf, zf))
    idx_ref[...] = oidx
    nxyz_ref[:, 0, :] = ox
    nxyz_ref[:, 1, :] = oy
    nxyz_ref[:, 2, :] = oz


def _fps(xyz_t):
    return pl.pallas_call(
        _fps_body,
        out_shape=[
            jax.ShapeDtypeStruct((B, NPOINT), jnp.int32),
            jax.ShapeDtypeStruct((B, 3, NPOINT), jnp.float32),
        ],
    )(xyz_t)


def _knn_body(xyz_ref, nxyz_ref, idx_ref, d_scr, i_scr):
    xmat = xyz_ref[0]
    cmat = nxyz_ref[0]
    mm = jnp.dot(xmat, cmat, preferred_element_type=jnp.float32)
    d = -2.0 * mm
    d = d + jnp.sum(xmat * xmat, axis=1, keepdims=True)
    d = d + jnp.sum(cmat * cmat, axis=0, keepdims=True)
    d_scr[...] = d
    iota = jax.lax.broadcasted_iota(jnp.int32, (N, CT), 0)

    def ext(k, carry):
        mprev, aprev = carry
        dv = d_scr[...]
        valid = (dv > mprev) | ((dv == mprev) & (iota > aprev))
        dm = jnp.where(valid, dv, BIGF)
        m = jnp.min(dm, axis=0, keepdims=True)
        am = jnp.min(jnp.where(dm == m, iota, N), axis=0,
                     keepdims=True).astype(jnp.int32)
        i_scr[pl.ds(k, 1), :] = am
        return m, am

    jax.lax.fori_loop(
        0, NSAMPLE, ext,
        (jnp.full((1, CT), -BIGF, jnp.float32),
         jnp.full((1, CT), -1, jnp.int32)))
    off = pl.program_id(0) * N
    idx_ref[0] = jnp.transpose(i_scr[...], (1, 0)) + off


def _knn(xyz, nxyz_b):
    return pl.pallas_call(
        _knn_body,
        grid=(B, NPOINT // CT),
        in_specs=[
            pl.BlockSpec((1, N, 3), lambda b, t: (b, 0, 0)),
            pl.BlockSpec((1, 3, CT), lambda b, t: (b, 0, t)),
        ],
        out_specs=pl.BlockSpec((1, CT, NSAMPLE), lambda b, t: (b, t, 0)),
        out_shape=jax.ShapeDtypeStruct((B, NPOINT, NSAMPLE), jnp.int32),
        scratch_shapes=[pltpu.VMEM((N, CT), jnp.float32),
                        pltpu.VMEM((NSAMPLE, CT), jnp.int32)],
    )(xyz, nxyz_b)


def _h_body(f_ref, x_ref, wf_ref, wx_ref, h_ref):
    f = f_ref[0]
    h = jax.lax.dot_general(
        f, wf_ref[...], (((0,), (0,)), ((), ())),
        preferred_element_type=jnp.float32)
    h = h + jnp.dot(x_ref[0], wx_ref[...],
                    preferred_element_type=jnp.float32)
    h_ref[0] = h


def _h_transform(features, xyz, w0ft, w0xt):
    return pl.pallas_call(
        _h_body,
        grid=(B, N // 512),
        in_specs=[
            pl.BlockSpec((1, CIN, 512), lambda b, t: (b, 0, t)),
            pl.BlockSpec((1, 512, 3), lambda b, t: (b, t, 0)),
            pl.BlockSpec((CIN, CIN), lambda b, t: (0, 0)),
            pl.BlockSpec((3, CIN), lambda b, t: (0, 0)),
        ],
        out_specs=pl.BlockSpec((1, 512, CIN), lambda b, t: (b, t, 0)),
        out_shape=jax.ShapeDtypeStruct((B, N, CIN), jnp.float32),
    )(features, xyz, w0ft, w0xt)


def _gather_sc(idx_flat, htab):
    mesh = plsc.VectorSubcoreMesh(core_axis_name="c", subcore_axis_name="s")

    nbuf = 4

    @functools.partial(
        pl.kernel, mesh=mesh,
        out_type=jax.ShapeDtypeStruct((PN, CIN), jnp.float32),
        scratch_types=(
            [pltpu.VMEM((PERW,), jnp.int32)]
            + [pltpu.VMEM((CH, CIN), jnp.float32)] * nbuf
            + [pltpu.SemaphoreType.DMA] * (2 * nbuf)
        ),
    )
    def k(idx_hbm, h_hbm, out_hbm, idxb, *bufs):
        grows = bufs[:nbuf]
        gsems = bufs[nbuf:2 * nbuf]
        wsems = bufs[2 * nbuf:]
        wid = lax.axis_index("s") * 2 + lax.axis_index("c")
        base = wid * PERW
        pltpu.sync_copy(idx_hbm.at[pl.ds(base, PERW)], idxb)

        def gth(q, _):
            offs = [pl.multiple_of((q * nbuf + i) * CH, CH)
                    for i in range(nbuf)]
            cps = [pltpu.async_copy(h_hbm.at[idxb.at[pl.ds(offs[i], CH)]],
                                    grows[i], gsems[i])
                   for i in range(nbuf)]
            wps = []
            for i in range(nbuf):
                cps[i].wait()
                wps.append(pltpu.async_copy(
                    grows[i], out_hbm.at[pl.ds(base + offs[i], CH)],
                    wsems[i]))
            for w in wps:
                w.wait()
            return 0

        jax.lax.fori_loop(0, PERW // CH // nbuf, gth, 0)

    return k(idx_flat, htab)


def _acc_stats(y, s_ref, q_ref):
    ps = jnp.sum(y, axis=0, keepdims=True)
    pq = jnp.sum(y * y, axis=0, keepdims=True)

    @pl.when(pl.program_id(0) == 0)
    def _():
        s_ref[...] = ps
        q_ref[...] = pq

    @pl.when(pl.program_id(0) != 0)
    def _():
        s_ref[...] = s_ref[...] + ps
        q_ref[...] = q_ref[...] + pq


def _mlp0_body(hg_ref, c_ref, wx_ref, b_ref, y_ref, s_ref, q_ref):
    cp = jnp.dot(c_ref[...], wx_ref[...],
                 preferred_element_type=jnp.float32)
    crep = jnp.broadcast_to(cp[:, None, :],
                            (TM // NSAMPLE, NSAMPLE, cp.shape[-1]))
    crep = crep.reshape(TM, cp.shape[-1])
    y = hg_ref[...] - crep
    y = y + b_ref[...]
    y_ref[...] = y
    _acc_stats(y, s_ref, q_ref)


def _norm_relu(y, s_ref, q_ref, g_ref, be_ref):
    mean = s_ref[...] / PN
    var = q_ref[...] / PN - mean * mean
    xn = (y - mean) / jnp.sqrt(var + EPS) * g_ref[...] + be_ref[...]
    return jnp.maximum(xn, 0.0)


def _mlp_mid_body(y0_ref, s0_ref, q0_ref, g_ref, be_ref, w_ref, b_ref,
                  y_ref, s_ref, q_ref):
    x = _norm_relu(y0_ref[...], s0_ref, q0_ref, g_ref, be_ref)
    y = jnp.dot(x, w_ref[...], preferred_element_type=jnp.float32) + b_ref[...]
    y_ref[...] = y
    _acc_stats(y, s_ref, q_ref)


def _mlp2_body(y1_ref, s1_ref, q1_ref, g_ref, be_ref, w_ref, b_ref,
               ymx_ref, ymn_ref, s_ref, q_ref):
    x = _norm_relu(y1_ref[...], s1_ref, q1_ref, g_ref, be_ref)
    y = jnp.dot(x, w_ref[...], preferred_element_type=jnp.float32) + b_ref[...]
    _acc_stats(y, s_ref, q_ref)
    yr = y.reshape(TM // NSAMPLE, NSAMPLE, y.shape[-1])
    ymx_ref[...] = jnp.max(yr, axis=1)
    ymn_ref[...] = jnp.min(yr, axis=1)


def _fin_body(ymx_ref, ymn_ref, s_ref, q_ref, g_ref, be_ref, o_ref):
    mean = s_ref[...] / PN
    var = q_ref[...] / PN - mean * mean
    g = g_ref[...]
    ysel = jnp.where(g >= 0, ymx_ref[...], ymn_ref[...])
    xn = (ysel - mean) / jnp.sqrt(var + EPS) * g + be_ref[...]
    o_ref[...] = jnp.maximum(xn, 0.0)


def _row_spec(c):
    return pl.BlockSpec((TM, c), lambda s: (s, 0))


def _full_spec(shape):
    return pl.BlockSpec(shape, lambda s: tuple(0 for _ in shape))


def _mlp(hg, cpad, params):
    (w0, b0, g0, be0), (w1, b1, g1, be1), (w2, b2, g2, be2) = params
    steps = PN // TM
    c1, c2 = 128, 256
    w0xp = jnp.zeros((8, c1), jnp.float32).at[:3, :].set(w0[:, :3].T)
    w1t = jnp.transpose(w1)
    w2t = jnp.transpose(w2)
    r = lambda v: v.reshape(1, -1)

    y0, s0, q0 = pl.pallas_call(
        _mlp0_body,
        grid=(steps,),
        in_specs=[
            _row_spec(CIN),
            pl.BlockSpec((TM // NSAMPLE, 8), lambda s: (s, 0)),
            _full_spec((8, c1)), _full_spec((1, c1)),
        ],
        out_specs=[
            _row_spec(c1),
            pl.BlockSpec((1, c1), lambda s: (0, 0)),
            pl.BlockSpec((1, c1), lambda s: (0, 0)),
        ],
        out_shape=[
            jax.ShapeDtypeStruct((PN, c1), jnp.float32),
            jax.ShapeDtypeStruct((1, c1), jnp.float32),
            jax.ShapeDtypeStruct((1, c1), jnp.float32),
        ],
    )(hg, cpad, w0xp, r(b0))

    def mid(y, s, q, g, be, wt, b, cout):
        return pl.pallas_call(
            _mlp_mid_body,
            grid=(steps,),
            in_specs=[
                _row_spec(y.shape[-1]),
                _full_spec((1, y.shape[-1])), _full_spec((1, y.shape[-1])),
                _full_spec((1, y.shape[-1])), _full_spec((1, y.shape[-1])),
                _full_spec((y.shape[-1], cout)), _full_spec((1, cout)),
            ],
            out_specs=[
                _row_spec(cout),
                pl.BlockSpec((1, cout), lambda s: (0, 0)),
                pl.BlockSpec((1, cout), lambda s: (0, 0)),
            ],
            out_shape=[
                jax.ShapeDtypeStruct((PN, cout), jnp.float32),
                jax.ShapeDtypeStruct((1, cout), jnp.float32),
                jax.ShapeDtypeStruct((1, cout), jnp.float32),
            ],
        )(y, s, q, r(g), r(be), wt, b)

    y1, s1, q1 = mid(y0, s0, q0, g0, be0, w1t, r(b1), c1)

    grp = TM // NSAMPLE
    ymx, ymn, s2, q2 = pl.pallas_call(
        _mlp2_body,
        grid=(steps,),
        in_specs=[
            _row_spec(c1),
            _full_spec((1, c1)), _full_spec((1, c1)),
            _full_spec((1, c1)), _full_spec((1, c1)),
            _full_spec((c1, c2)), _full_spec((1, c2)),
        ],
        out_specs=[
            pl.BlockSpec((grp, c2), lambda s: (s, 0)),
            pl.BlockSpec((grp, c2), lambda s: (s, 0)),
            pl.BlockSpec((1, c2), lambda s: (0, 0)),
            pl.BlockSpec((1, c2), lambda s: (0, 0)),
        ],
        out_shape=[
            jax.ShapeDtypeStruct((B * NPOINT, c2), jnp.float32),
            jax.ShapeDtypeStruct((B * NPOINT, c2), jnp.float32),
            jax.ShapeDtypeStruct((1, c2), jnp.float32),
            jax.ShapeDtypeStruct((1, c2), jnp.float32),
        ],
    )(y1, s1, q1, r(g1), r(be1), w2t, r(b2))

    FT = 2048
    out = pl.pallas_call(
        _fin_body,
        grid=(B * NPOINT // FT,),
        in_specs=[
            pl.BlockSpec((FT, c2), lambda s: (s, 0)),
            pl.BlockSpec((FT, c2), lambda s: (s, 0)),
            _full_spec((1, c2)), _full_spec((1, c2)),
            _full_spec((1, c2)), _full_spec((1, c2)),
        ],
        out_specs=pl.BlockSpec((FT, c2), lambda s: (s, 0)),
        out_shape=jax.ShapeDtypeStruct((B * NPOINT, c2), jnp.float32),
    )(ymx, ymn, s2, q2, r(g2), r(be2))
    return out


def kernel(xyz, features, W0, b0, g0, be0, W1, b1, g1, be1, W2, b2, g2, be2):
    xyz_t = jnp.transpose(xyz, (2, 0, 1))
    _, nxyz_b = _fps(xyz_t)
    new_xyz = jnp.transpose(nxyz_b, (0, 2, 1))

    htab = _h_transform(features, xyz, jnp.transpose(W0[:, 3:]),
                        jnp.transpose(W0[:, :3])).reshape(B * N, CIN)

    idx = _knn(xyz, nxyz_b)
    hg = _gather_sc(idx.reshape(PN), htab)

    cpad = jnp.pad(new_xyz, ((0, 0), (0, 0), (0, 5))).reshape(B * NPOINT, 8)
    params = [(W0, b0, g0, be0), (W1, b1, g1, be1), (W2, b2, g2, be2)]
    outf = _mlp(hg, cpad, params)
    new_features = jnp.transpose(outf.reshape(B, NPOINT, 256), (0, 2, 1))
    return new_xyz, new_features


# speedup vs baseline: 1.1229x; 1.0383x over previous
import functools

import jax
import jax.numpy as jnp
from jax import lax
from jax.experimental import pallas as pl
from jax.experimental.pallas import tpu as pltpu
from jax.experimental.pallas import tpu_sc as plsc

B = 8
N = 4096
NPOINT = 1024
NSAMPLE = 32
CIN = 128
EPS = 1e-5
BIGF = 1e10
CT = 256
TM = 4096
PN = B * NPOINT * NSAMPLE
NW = 32
PERW = PN // NW
CH = 128


def _fps_body(xyz_ref, idx_ref, nxyz_ref):
    xs = xyz_ref[0]
    ys = xyz_ref[1]
    zs = xyz_ref[2]
    iota = jax.lax.broadcasted_iota(jnp.int32, (B, N), 1)
    row_iota = jax.lax.broadcasted_iota(jnp.int32, (B, NPOINT), 0)
    iota_np = jax.lax.broadcasted_iota(jnp.int32, (B, NPOINT), 1)

    def body(i, carry):
        dist, far, oidx, ox, oy, oz = carry
        oh = iota == far
        cx = jnp.sum(jnp.where(oh, xs, 0.0), axis=1, keepdims=True)
        cy = jnp.sum(jnp.where(oh, ys, 0.0), axis=1, keepdims=True)
        cz = jnp.sum(jnp.where(oh, zs, 0.0), axis=1, keepdims=True)
        sel = (iota_np == i) & (row_iota >= 0)
        oidx = oidx + jnp.where(sel, jnp.broadcast_to(far, (B, NPOINT)), 0)
        ox = ox + jnp.where(sel, jnp.broadcast_to(cx, (B, NPOINT)), 0.0)
        oy = oy + jnp.where(sel, jnp.broadcast_to(cy, (B, NPOINT)), 0.0)
        oz = oz + jnp.where(sel, jnp.broadcast_to(cz, (B, NPOINT)), 0.0)
        d = (xs - cx) ** 2 + (ys - cy) ** 2 + (zs - cz) ** 2
        dist = jnp.minimum(dist, d)
        m = jnp.max(dist, axis=1, keepdims=True)
        far2 = jnp.min(jnp.where(dist == m, iota, N), axis=1,
                       keepdims=True).astype(jnp.int32)
        return dist, far2, oidx, ox, oy, oz

    dist0 = jnp.full((B, N), BIGF, jnp.float32)
    far0 = jnp.zeros((B, 1), jnp.int32)
    zf = jnp.zeros((B, NPOINT), jnp.float32)
    zi = jnp.zeros((B, NPOINT), jnp.int32)
    _, _, oidx, ox, oy, oz = jax.lax.fori_loop(
        0, NPOINT, body, (dist0, far0, zi, zf, zf, zf))
    idx_ref[...] = oidx
    nxyz_ref[:, 0, :] = ox
    nxyz_ref[:, 1, :] = oy
    nxyz_ref[:, 2, :] = oz


def _fps(xyz_t):
    return pl.pallas_call(
        _fps_body,
        out_shape=[
            jax.ShapeDtypeStruct((B, NPOINT), jnp.int32),
            jax.ShapeDtypeStruct((B, 3, NPOINT), jnp.float32),
        ],
    )(xyz_t)


def _knn_body(xyz_ref, nxyz_ref, idx_ref, d_scr, i_scr):
    xmat = xyz_ref[0]
    cmat = nxyz_ref[0]
    mm = jnp.dot(xmat, cmat, preferred_element_type=jnp.float32)
    d = -2.0 * mm
    d = d + jnp.sum(xmat * xmat, axis=1, keepdims=True)
    d = d + jnp.sum(cmat * cmat, axis=0, keepdims=True)
    d_scr[...] = d
    iota = jax.lax.broadcasted_iota(jnp.int32, (N, CT), 0)

    def ext(k, carry):
        mprev, aprev = carry
        dv = d_scr[...]
        valid = (dv > mprev) | ((dv == mprev) & (iota > aprev))
        dm = jnp.where(valid, dv, BIGF)
        m = jnp.min(dm, axis=0, keepdims=True)
        am = jnp.min(jnp.where(dm == m, iota, N), axis=0,
                     keepdims=True).astype(jnp.int32)
        i_scr[pl.ds(k, 1), :] = am
        return m, am

    jax.lax.fori_loop(
        0, NSAMPLE, ext,
        (jnp.full((1, CT), -BIGF, jnp.float32),
         jnp.full((1, CT), -1, jnp.int32)))
    off = pl.program_id(0) * N
    idx_ref[0] = jnp.transpose(i_scr[...], (1, 0)) + off


def _knn(xyz, nxyz_b):
    return pl.pallas_call(
        _knn_body,
        grid=(B, NPOINT // CT),
        in_specs=[
            pl.BlockSpec((1, N, 3), lambda b, t: (b, 0, 0)),
            pl.BlockSpec((1, 3, CT), lambda b, t: (b, 0, t)),
        ],
        out_specs=pl.BlockSpec((1, CT, NSAMPLE), lambda b, t: (b, t, 0)),
        out_shape=jax.ShapeDtypeStruct((B, NPOINT, NSAMPLE), jnp.int32),
        scratch_shapes=[pltpu.VMEM((N, CT), jnp.float32),
                        pltpu.VMEM((NSAMPLE, CT), jnp.int32)],
    )(xyz, nxyz_b)


def _h_body(f_ref, x_ref, wf_ref, wx_ref, h_ref):
    f = f_ref[0]
    h = jax.lax.dot_general(
        f, wf_ref[...], (((0,), (0,)), ((), ())),
        preferred_element_type=jnp.float32)
    h = h + jnp.dot(x_ref[0], wx_ref[...],
                    preferred_element_type=jnp.float32)
    h_ref[0] = h


def _h_transform(features, xyz, w0ft, w0xt):
    return pl.pallas_call(
        _h_body,
        grid=(B, N // 512),
        in_specs=[
            pl.BlockSpec((1, CIN, 512), lambda b, t: (b, 0, t)),
            pl.BlockSpec((1, 512, 3), lambda b, t: (b, t, 0)),
            pl.BlockSpec((CIN, CIN), lambda b, t: (0, 0)),
            pl.BlockSpec((3, CIN), lambda b, t: (0, 0)),
        ],
        out_specs=pl.BlockSpec((1, 512, CIN), lambda b, t: (b, t, 0)),
        out_shape=jax.ShapeDtypeStruct((B, N, CIN), jnp.float32),
    )(features, xyz, w0ft, w0xt)


def _gather_sc(idx_flat, htab):
    mesh = plsc.VectorSubcoreMesh(core_axis_name="c", subcore_axis_name="s")

    nbuf = 4

    @functools.partial(
        pl.kernel, mesh=mesh,
        out_type=jax.ShapeDtypeStruct((PN, CIN), jnp.float32),
        scratch_types=(
            [pltpu.VMEM((PERW,), jnp.int32)]
            + [pltpu.VMEM((CH, CIN), jnp.float32)] * nbuf
            + [pltpu.SemaphoreType.DMA] * (2 * nbuf)
        ),
    )
    def k(idx_hbm, h_hbm, out_hbm, idxb, *bufs):
        grows = bufs[:nbuf]
        gsems = bufs[nbuf:2 * nbuf]
        wsems = bufs[2 * nbuf:]
        wid = lax.axis_index("s") * 2 + lax.axis_index("c")
        base = wid * PERW
        pltpu.sync_copy(idx_hbm.at[pl.ds(base, PERW)], idxb)

        def gth(q, _):
            offs = [pl.multiple_of((q * nbuf + i) * CH, CH)
                    for i in range(nbuf)]
            cps = [pltpu.async_copy(h_hbm.at[idxb.at[pl.ds(offs[i], CH)]],
                                    grows[i], gsems[i])
                   for i in range(nbuf)]
            wps = []
            for i in range(nbuf):
                cps[i].wait()
                wps.append(pltpu.async_copy(
                    grows[i], out_hbm.at[pl.ds(base + offs[i], CH)],
                    wsems[i]))
            for w in wps:
                w.wait()
            return 0

        jax.lax.fori_loop(0, PERW // CH // nbuf, gth, 0)

    return k(idx_flat, htab)


def _acc_stats(y, s_ref, q_ref):
    ps = jnp.sum(y, axis=0, keepdims=True)
    pq = jnp.sum(y * y, axis=0, keepdims=True)

    @pl.when(pl.program_id(0) == 0)
    def _():
        s_ref[...] = ps
        q_ref[...] = pq

    @pl.when(pl.program_id(0) != 0)
    def _():
        s_ref[...] = s_ref[...] + ps
        q_ref[...] = q_ref[...] + pq


def _mlp0_body(hg_ref, c_ref, wx_ref, b_ref, y_ref, s_ref, q_ref):
    cp = jnp.dot(c_ref[...], wx_ref[...],
                 preferred_element_type=jnp.float32)
    crep = jnp.broadcast_to(cp[:, None, :],
                            (TM // NSAMPLE, NSAMPLE, cp.shape[-1]))
    crep = crep.reshape(TM, cp.shape[-1])
    y = hg_ref[...] - crep
    y = y + b_ref[...]
    y_ref[...] = y
    _acc_stats(y, s_ref, q_ref)


def _norm_relu(y, s_ref, q_ref, g_ref, be_ref):
    mean = s_ref[...] / PN
    var = q_ref[...] / PN - mean * mean
    xn = (y - mean) / jnp.sqrt(var + EPS) * g_ref[...] + be_ref[...]
    return jnp.maximum(xn, 0.0)


def _mlp_mid_body(y0_ref, s0_ref, q0_ref, g_ref, be_ref, w_ref, b_ref,
                  y_ref, s_ref, q_ref):
    x = _norm_relu(y0_ref[...], s0_ref, q0_ref, g_ref, be_ref)
    y = jnp.dot(x, w_ref[...], preferred_element_type=jnp.float32) + b_ref[...]
    y_ref[...] = y
    _acc_stats(y, s_ref, q_ref)


def _mlp2_body(y1_ref, s1_ref, q1_ref, g_ref, be_ref, w_ref, b_ref,
               ymx_ref, ymn_ref, s_ref, q_ref):
    x = _norm_relu(y1_ref[...], s1_ref, q1_ref, g_ref, be_ref)
    y = jnp.dot(x, w_ref[...], preferred_element_type=jnp.float32) + b_ref[...]
    _acc_stats(y, s_ref, q_ref)
    yr = y.reshape(TM // NSAMPLE, NSAMPLE, y.shape[-1])
    ymx_ref[...] = jnp.max(yr, axis=1)
    ymn_ref[...] = jnp.min(yr, axis=1)


def _fin_body(ymx_ref, ymn_ref, s_ref, q_ref, g_ref, be_ref, o_ref):
    mean = s_ref[...] / PN
    var = q_ref[...] / PN - mean * mean
    g = g_ref[...]
    ysel = jnp.where(g >= 0, ymx_ref[...], ymn_ref[...])
    xn = (ysel - mean) / jnp.sqrt(var + EPS) * g + be_ref[...]
    o_ref[...] = jnp.maximum(xn, 0.0)


def _row_spec(c):
    return pl.BlockSpec((TM, c), lambda s: (s, 0))


def _full_spec(shape):
    return pl.BlockSpec(shape, lambda s: tuple(0 for _ in shape))


def _mlp(hg, cpad, params):
    (w0, b0, g0, be0), (w1, b1, g1, be1), (w2, b2, g2, be2) = params
    steps = PN // TM
    c1, c2 = 128, 256
    w0xp = jnp.zeros((8, c1), jnp.float32).at[:3, :].set(w0[:, :3].T)
    w1t = jnp.transpose(w1)
    w2t = jnp.transpose(w2)
    r = lambda v: v.reshape(1, -1)

    y0, s0, q0 = pl.pallas_call(
        _mlp0_body,
        grid=(steps,),
        in_specs=[
            _row_spec(CIN),
            pl.BlockSpec((TM // NSAMPLE, 8), lambda s: (s, 0)),
            _full_spec((8, c1)), _full_spec((1, c1)),
        ],
        out_specs=[
            _row_spec(c1),
            pl.BlockSpec((1, c1), lambda s: (0, 0)),
            pl.BlockSpec((1, c1), lambda s: (0, 0)),
        ],
        out_shape=[
            jax.ShapeDtypeStruct((PN, c1), jnp.float32),
            jax.ShapeDtypeStruct((1, c1), jnp.float32),
            jax.ShapeDtypeStruct((1, c1), jnp.float32),
        ],
    )(hg, cpad, w0xp, r(b0))

    def mid(y, s, q, g, be, wt, b, cout):
        return pl.pallas_call(
            _mlp_mid_body,
            grid=(steps,),
            in_specs=[
                _row_spec(y.shape[-1]),
                _full_spec((1, y.shape[-1])), _full_spec((1, y.shape[-1])),
                _full_spec((1, y.shape[-1])), _full_spec((1, y.shape[-1])),
                _full_spec((y.shape[-1], cout)), _full_spec((1, cout)),
            ],
            out_specs=[
                _row_spec(cout),
                pl.BlockSpec((1, cout), lambda s: (0, 0)),
                pl.BlockSpec((1, cout), lambda s: (0, 0)),
            ],
            out_shape=[
                jax.ShapeDtypeStruct((PN, cout), jnp.float32),
                jax.ShapeDtypeStruct((1, cout), jnp.float32),
                jax.ShapeDtypeStruct((1, cout), jnp.float32),
            ],
        )(y, s, q, r(g), r(be), wt, b)

    y1, s1, q1 = mid(y0, s0, q0, g0, be0, w1t, r(b1), c1)

    grp = TM // NSAMPLE
    ymx, ymn, s2, q2 = pl.pallas_call(
        _mlp2_body,
        grid=(steps,),
        in_specs=[
            _row_spec(c1),
            _full_spec((1, c1)), _full_spec((1, c1)),
            _full_spec((1, c1)), _full_spec((1, c1)),
            _full_spec((c1, c2)), _full_spec((1, c2)),
        ],
        out_specs=[
            pl.BlockSpec((grp, c2), lambda s: (s, 0)),
            pl.BlockSpec((grp, c2), lambda s: (s, 0)),
            pl.BlockSpec((1, c2), lambda s: (0, 0)),
            pl.BlockSpec((1, c2), lambda s: (0, 0)),
        ],
        out_shape=[
            jax.ShapeDtypeStruct((B * NPOINT, c2), jnp.float32),
            jax.ShapeDtypeStruct((B * NPOINT, c2), jnp.float32),
            jax.ShapeDtypeStruct((1, c2), jnp.float32),
            jax.ShapeDtypeStruct((1, c2), jnp.float32),
        ],
    )(y1, s1, q1, r(g1), r(be1), w2t, r(b2))

    FT = 2048
    out = pl.pallas_call(
        _fin_body,
        grid=(B * NPOINT // FT,),
        in_specs=[
            pl.BlockSpec((FT, c2), lambda s: (s, 0)),
            pl.BlockSpec((FT, c2), lambda s: (s, 0)),
            _full_spec((1, c2)), _full_spec((1, c2)),
            _full_spec((1, c2)), _full_spec((1, c2)),
        ],
        out_specs=pl.BlockSpec((FT, c2), lambda s: (s, 0)),
        out_shape=jax.ShapeDtypeStruct((B * NPOINT, c2), jnp.float32),
    )(ymx, ymn, s2, q2, r(g2), r(be2))
    return out


def kernel(xyz, features, W0, b0, g0, be0, W1, b1, g1, be1, W2, b2, g2, be2):
    xyz_t = jnp.transpose(xyz, (2, 0, 1))
    _, nxyz_b = _fps(xyz_t)
    new_xyz = jnp.transpose(nxyz_b, (0, 2, 1))

    htab = _h_transform(features, xyz, jnp.transpose(W0[:, 3:]),
                        jnp.transpose(W0[:, :3])).reshape(B * N, CIN)

    idx = _knn(xyz, nxyz_b)
    hg = _gather_sc(idx.reshape(PN), htab)

    cpad = jnp.pad(new_xyz, ((0, 0), (0, 0), (0, 5))).reshape(B * NPOINT, 8)
    params = [(W0, b0, g0, be0), (W1, b1, g1, be1), (W2, b2, g2, be2)]
    outf = _mlp(hg, cpad, params)
    new_features = jnp.transpose(outf.reshape(B, NPOINT, 256), (0, 2, 1))
    return new_xyz, new_features


# speedup vs baseline: 1.1373x; 1.0128x over previous
import functools

import jax
import jax.numpy as jnp
from jax import lax
from jax.experimental import pallas as pl
from jax.experimental.pallas import tpu as pltpu
from jax.experimental.pallas import tpu_sc as plsc

B = 8
N = 4096
NPOINT = 1024
NSAMPLE = 32
CIN = 128
EPS = 1e-5
BIGF = 1e10
CT = 256
TM = 8192
PN = B * NPOINT * NSAMPLE
NW = 32
PERW = PN // NW
CH = 128


def _fps_body(xyz_ref, idx_ref, nxyz_ref):
    xs = xyz_ref[0]
    ys = xyz_ref[1]
    zs = xyz_ref[2]
    iota = jax.lax.broadcasted_iota(jnp.int32, (B, N), 1)
    row_iota = jax.lax.broadcasted_iota(jnp.int32, (B, NPOINT), 0)
    iota_np = jax.lax.broadcasted_iota(jnp.int32, (B, NPOINT), 1)

    def body(i, carry):
        dist, far, oidx, ox, oy, oz = carry
        oh = iota == far
        cx = jnp.sum(jnp.where(oh, xs, 0.0), axis=1, keepdims=True)
        cy = jnp.sum(jnp.where(oh, ys, 0.0), axis=1, keepdims=True)
        cz = jnp.sum(jnp.where(oh, zs, 0.0), axis=1, keepdims=True)
        sel = (iota_np == i) & (row_iota >= 0)
        oidx = oidx + jnp.where(sel, jnp.broadcast_to(far, (B, NPOINT)), 0)
        ox = ox + jnp.where(sel, jnp.broadcast_to(cx, (B, NPOINT)), 0.0)
        oy = oy + jnp.where(sel, jnp.broadcast_to(cy, (B, NPOINT)), 0.0)
        oz = oz + jnp.where(sel, jnp.broadcast_to(cz, (B, NPOINT)), 0.0)
        d = (xs - cx) ** 2 + (ys - cy) ** 2 + (zs - cz) ** 2
        dist = jnp.minimum(dist, d)
        m = jnp.max(dist, axis=1, keepdims=True)
        far2 = jnp.min(jnp.where(dist == m, iota, N), axis=1,
                       keepdims=True).astype(jnp.int32)
        return dist, far2, oidx, ox, oy, oz

    dist0 = jnp.full((B, N), BIGF, jnp.float32)
    far0 = jnp.zeros((B, 1), jnp.int32)
    zf = jnp.zeros((B, NPOINT), jnp.float32)
    zi = jnp.zeros((B, NPOINT), jnp.int32)
    _, _, oidx, ox, oy, oz = jax.lax.fori_loop(
        0, NPOINT, body, (dist0, far0, zi, zf, zf, zf))
    idx_ref[...] = oidx
    nxyz_ref[:, 0, :] = ox
    nxyz_ref[:, 1, :] = oy
    nxyz_ref[:, 2, :] = oz


def _fps(xyz_t):
    return pl.pallas_call(
        _fps_body,
        out_shape=[
            jax.ShapeDtypeStruct((B, NPOINT), jnp.int32),
            jax.ShapeDtypeStruct((B, 3, NPOINT), jnp.float32),
        ],
    )(xyz_t)


def _knn_body(xyz_ref, nxyz_ref, idx_ref, d_scr, i_scr):
    xmat = xyz_ref[0]
    cmat = nxyz_ref[0]
    mm = jnp.dot(xmat, cmat, preferred_element_type=jnp.float32)
    d = -2.0 * mm
    d = d + jnp.sum(xmat * xmat, axis=1, keepdims=True)
    d = d + jnp.sum(cmat * cmat, axis=0, keepdims=True)
    d_scr[...] = d
    iota = jax.lax.broadcasted_iota(jnp.int32, (N, CT), 0)

    def ext(k, carry):
        mprev, aprev = carry
        dv = d_scr[...]
        valid = (dv > mprev) | ((dv == mprev) & (iota > aprev))
        dm = jnp.where(valid, dv, BIGF)
        m = jnp.min(dm, axis=0, keepdims=True)
        am = jnp.min(jnp.where(dm == m, iota, N), axis=0,
                     keepdims=True).astype(jnp.int32)
        i_scr[pl.ds(k, 1), :] = am
        return m, am

    jax.lax.fori_loop(
        0, NSAMPLE, ext,
        (jnp.full((1, CT), -BIGF, jnp.float32),
         jnp.full((1, CT), -1, jnp.int32)))
    off = pl.program_id(0) * N
    idx_ref[0] = jnp.transpose(i_scr[...], (1, 0)) + off


def _knn(xyz, nxyz_b):
    return pl.pallas_call(
        _knn_body,
        grid=(B, NPOINT // CT),
        in_specs=[
            pl.BlockSpec((1, N, 3), lambda b, t: (b, 0, 0)),
            pl.BlockSpec((1, 3, CT), lambda b, t: (b, 0, t)),
        ],
        out_specs=pl.BlockSpec((1, CT, NSAMPLE), lambda b, t: (b, t, 0)),
        out_shape=jax.ShapeDtypeStruct((B, NPOINT, NSAMPLE), jnp.int32),
        scratch_shapes=[pltpu.VMEM((N, CT), jnp.float32),
                        pltpu.VMEM((NSAMPLE, CT), jnp.int32)],
    )(xyz, nxyz_b)


def _h_body(f_ref, x_ref, wf_ref, wx_ref, h_ref):
    f = f_ref[0]
    h = jax.lax.dot_general(
        f, wf_ref[...], (((0,), (0,)), ((), ())),
        preferred_element_type=jnp.float32)
    h = h + jnp.dot(x_ref[0], wx_ref[...],
                    preferred_element_type=jnp.float32)
    h_ref[0] = h


def _h_transform(features, xyz, w0ft, w0xt):
    return pl.pallas_call(
        _h_body,
        grid=(B, N // 512),
        in_specs=[
            pl.BlockSpec((1, CIN, 512), lambda b, t: (b, 0, t)),
            pl.BlockSpec((1, 512, 3), lambda b, t: (b, t, 0)),
            pl.BlockSpec((CIN, CIN), lambda b, t: (0, 0)),
            pl.BlockSpec((3, CIN), lambda b, t: (0, 0)),
        ],
        out_specs=pl.BlockSpec((1, 512, CIN), lambda b, t: (b, t, 0)),
        out_shape=jax.ShapeDtypeStruct((B, N, CIN), jnp.float32),
    )(features, xyz, w0ft, w0xt)


def _gather_sc(idx_flat, htab):
    mesh = plsc.VectorSubcoreMesh(core_axis_name="c", subcore_axis_name="s")

    nbuf = 4

    @functools.partial(
        pl.kernel, mesh=mesh,
        out_type=jax.ShapeDtypeStruct((PN, CIN), jnp.float32),
        scratch_types=(
            [pltpu.VMEM((PERW,), jnp.int32)]
            + [pltpu.VMEM((CH, CIN), jnp.float32)] * nbuf
            + [pltpu.SemaphoreType.DMA] * (2 * nbuf)
        ),
    )
    def k(idx_hbm, h_hbm, out_hbm, idxb, *bufs):
        grows = bufs[:nbuf]
        gsems = bufs[nbuf:2 * nbuf]
        wsems = bufs[2 * nbuf:]
        wid = lax.axis_index("s") * 2 + lax.axis_index("c")
        base = wid * PERW
        pltpu.sync_copy(idx_hbm.at[pl.ds(base, PERW)], idxb)

        def gth(q, _):
            offs = [pl.multiple_of((q * nbuf + i) * CH, CH)
                    for i in range(nbuf)]
            cps = [pltpu.async_copy(h_hbm.at[idxb.at[pl.ds(offs[i], CH)]],
                                    grows[i], gsems[i])
                   for i in range(nbuf)]
            wps = []
            for i in range(nbuf):
                cps[i].wait()
                wps.append(pltpu.async_copy(
                    grows[i], out_hbm.at[pl.ds(base + offs[i], CH)],
                    wsems[i]))
            for w in wps:
                w.wait()
            return 0

        jax.lax.fori_loop(0, PERW // CH // nbuf, gth, 0)

    return k(idx_flat, htab)


def _acc_stats(y, s_ref, q_ref):
    ps = jnp.sum(y, axis=0, keepdims=True)
    pq = jnp.sum(y * y, axis=0, keepdims=True)

    @pl.when(pl.program_id(0) == 0)
    def _():
        s_ref[...] = ps
        q_ref[...] = pq

    @pl.when(pl.program_id(0) != 0)
    def _():
        s_ref[...] = s_ref[...] + ps
        q_ref[...] = q_ref[...] + pq


def _mlp0_body(hg_ref, c_ref, wx_ref, b_ref, y_ref, s_ref, q_ref):
    cp = jnp.dot(c_ref[...], wx_ref[...],
                 preferred_element_type=jnp.float32)
    crep = jnp.broadcast_to(cp[:, None, :],
                            (TM // NSAMPLE, NSAMPLE, cp.shape[-1]))
    crep = crep.reshape(TM, cp.shape[-1])
    y = hg_ref[...] - crep
    y = y + b_ref[...]
    y_ref[...] = y
    _acc_stats(y, s_ref, q_ref)


def _norm_relu(y, s_ref, q_ref, g_ref, be_ref):
    mean = s_ref[...] / PN
    var = q_ref[...] / PN - mean * mean
    xn = (y - mean) / jnp.sqrt(var + EPS) * g_ref[...] + be_ref[...]
    return jnp.maximum(xn, 0.0)


def _mlp_mid_body(y0_ref, s0_ref, q0_ref, g_ref, be_ref, w_ref, b_ref,
                  y_ref, s_ref, q_ref):
    x = _norm_relu(y0_ref[...], s0_ref, q0_ref, g_ref, be_ref)
    y = jnp.dot(x, w_ref[...], preferred_element_type=jnp.float32) + b_ref[...]
    y_ref[...] = y
    _acc_stats(y, s_ref, q_ref)


def _mlp2_body(y1_ref, s1_ref, q1_ref, g_ref, be_ref, w_ref, b_ref,
               ymx_ref, ymn_ref, s_ref, q_ref):
    x = _norm_relu(y1_ref[...], s1_ref, q1_ref, g_ref, be_ref)
    y = jnp.dot(x, w_ref[...], preferred_element_type=jnp.float32) + b_ref[...]
    _acc_stats(y, s_ref, q_ref)
    yr = y.reshape(TM // NSAMPLE, NSAMPLE, y.shape[-1])
    ymx_ref[...] = jnp.max(yr, axis=1)
    ymn_ref[...] = jnp.min(yr, axis=1)


def _fin_body(ymx_ref, ymn_ref, s_ref, q_ref, g_ref, be_ref, o_ref):
    mean = s_ref[...] / PN
    var = q_ref[...] / PN - mean * mean
    g = g_ref[...]
    ysel = jnp.where(g >= 0, ymx_ref[...], ymn_ref[...])
    xn = (ysel - mean) / jnp.sqrt(var + EPS) * g + be_ref[...]
    o_ref[...] = jnp.maximum(xn, 0.0)


def _row_spec(c):
    return pl.BlockSpec((TM, c), lambda s: (s, 0))


def _full_spec(shape):
    return pl.BlockSpec(shape, lambda s: tuple(0 for _ in shape))


def _mlp(hg, cpad, params):
    (w0, b0, g0, be0), (w1, b1, g1, be1), (w2, b2, g2, be2) = params
    steps = PN // TM
    c1, c2 = 128, 256
    w0xp = jnp.zeros((8, c1), jnp.float32).at[:3, :].set(w0[:, :3].T)
    w1t = jnp.transpose(w1)
    w2t = jnp.transpose(w2)
    r = lambda v: v.reshape(1, -1)

    y0, s0, q0 = pl.pallas_call(
        _mlp0_body,
        grid=(steps,),
        in_specs=[
            _row_spec(CIN),
            pl.BlockSpec((TM // NSAMPLE, 8), lambda s: (s, 0)),
            _full_spec((8, c1)), _full_spec((1, c1)),
        ],
        out_specs=[
            _row_spec(c1),
            pl.BlockSpec((1, c1), lambda s: (0, 0)),
            pl.BlockSpec((1, c1), lambda s: (0, 0)),
        ],
        out_shape=[
            jax.ShapeDtypeStruct((PN, c1), jnp.float32),
            jax.ShapeDtypeStruct((1, c1), jnp.float32),
            jax.ShapeDtypeStruct((1, c1), jnp.float32),
        ],
    )(hg, cpad, w0xp, r(b0))

    def mid(y, s, q, g, be, wt, b, cout):
        return pl.pallas_call(
            _mlp_mid_body,
            grid=(steps,),
            in_specs=[
                _row_spec(y.shape[-1]),
                _full_spec((1, y.shape[-1])), _full_spec((1, y.shape[-1])),
                _full_spec((1, y.shape[-1])), _full_spec((1, y.shape[-1])),
                _full_spec((y.shape[-1], cout)), _full_spec((1, cout)),
            ],
            out_specs=[
                _row_spec(cout),
                pl.BlockSpec((1, cout), lambda s: (0, 0)),
                pl.BlockSpec((1, cout), lambda s: (0, 0)),
            ],
            out_shape=[
                jax.ShapeDtypeStruct((PN, cout), jnp.float32),
                jax.ShapeDtypeStruct((1, cout), jnp.float32),
                jax.ShapeDtypeStruct((1, cout), jnp.float32),
            ],
        )(y, s, q, r(g), r(be), wt, b)

    y1, s1, q1 = mid(y0, s0, q0, g0, be0, w1t, r(b1), c1)

    grp = TM // NSAMPLE
    ymx, ymn, s2, q2 = pl.pallas_call(
        _mlp2_body,
        grid=(steps,),
        in_specs=[
            _row_spec(c1),
            _full_spec((1, c1)), _full_spec((1, c1)),
            _full_spec((1, c1)), _full_spec((1, c1)),
            _full_spec((c1, c2)), _full_spec((1, c2)),
        ],
        out_specs=[
            pl.BlockSpec((grp, c2), lambda s: (s, 0)),
            pl.BlockSpec((grp, c2), lambda s: (s, 0)),
            pl.BlockSpec((1, c2), lambda s: (0, 0)),
            pl.BlockSpec((1, c2), lambda s: (0, 0)),
        ],
        out_shape=[
            jax.ShapeDtypeStruct((B * NPOINT, c2), jnp.float32),
            jax.ShapeDtypeStruct((B * NPOINT, c2), jnp.float32),
            jax.ShapeDtypeStruct((1, c2), jnp.float32),
            jax.ShapeDtypeStruct((1, c2), jnp.float32),
        ],
    )(y1, s1, q1, r(g1), r(be1), w2t, r(b2))

    FT = 2048
    out = pl.pallas_call(
        _fin_body,
        grid=(B * NPOINT // FT,),
        in_specs=[
            pl.BlockSpec((FT, c2), lambda s: (s, 0)),
            pl.BlockSpec((FT, c2), lambda s: (s, 0)),
            _full_spec((1, c2)), _full_spec((1, c2)),
            _full_spec((1, c2)), _full_spec((1, c2)),
        ],
        out_specs=pl.BlockSpec((FT, c2), lambda s: (s, 0)),
        out_shape=jax.ShapeDtypeStruct((B * NPOINT, c2), jnp.float32),
    )(ymx, ymn, s2, q2, r(g2), r(be2))
    return out


def kernel(xyz, features, W0, b0, g0, be0, W1, b1, g1, be1, W2, b2, g2, be2):
    xyz_t = jnp.transpose(xyz, (2, 0, 1))
    _, nxyz_b = _fps(xyz_t)
    new_xyz = jnp.transpose(nxyz_b, (0, 2, 1))

    htab = _h_transform(features, xyz, jnp.transpose(W0[:, 3:]),
                        jnp.transpose(W0[:, :3])).reshape(B * N, CIN)

    idx = _knn(xyz, nxyz_b)
    hg = _gather_sc(idx.reshape(PN), htab)

    cpad = jnp.pad(new_xyz, ((0, 0), (0, 0), (0, 5))).reshape(B * NPOINT, 8)
    params = [(W0, b0, g0, be0), (W1, b1, g1, be1), (W2, b2, g2, be2)]
    outf = _mlp(hg, cpad, params)
    new_features = jnp.transpose(outf.reshape(B, NPOINT, 256), (0, 2, 1))
    return new_xyz, new_features
